# baseline (speedup 1.0000x reference)
import functools
import math

import jax
import jax.numpy as jnp
from jax import lax
from jax.experimental import pallas as pl
from jax.experimental.pallas import tpu as pltpu
from jax.experimental.pallas import tpu_sc as plsc

L = 16


def _signed_key(b):
    return b ^ (lax.shift_right_arithmetic(b, 31) & jnp.int32(0x7FFFFFFF))


def _matmul_t_body(x_ref, w_ref, o_ref):
    o_ref[...] = lax.dot_general(
        w_ref[...], x_ref[...], (((1,), (1,)), ((), ())),
        preferred_element_type=jnp.float32)


def _scores_t(x, w):
    B, D = x.shape
    O = w.shape[0]
    CB = 128
    return pl.pallas_call(
        _matmul_t_body,
        grid=(O // CB,),
        in_specs=[
            pl.BlockSpec((B, D), lambda j: (0, 0)),
            pl.BlockSpec((CB, D), lambda j: (j, 0)),
        ],
        out_specs=pl.BlockSpec((CB, B), lambda j: (j, 0)),
        out_shape=jax.ShapeDtypeStruct((O, B), jnp.float32),
    )(x, w)


def _walk(totals_v, rank):
    lane = lax.iota(jnp.int32, L)

    def body(g, carry):
        cc, dig, rin, bcnt = carry
        gg = 15 - g
        base = gg * L + lane
        T = plsc.load_gather(totals_v, [base])
        revT = lax.rev(T, (0,))
        csum = plsc.cumsum(revT)
        S = csum + cc
        m = S >= rank
        pc = jnp.sum(m.astype(jnp.int32))
        hit = jnp.logical_and(pc > 0, dig < 0)
        new_dig = gg * L + pc - 1
        above = cc + jnp.sum(jnp.where(m, 0, revT))
        first = jnp.logical_and(m, (S - revT) < rank)
        new_bcnt = jnp.sum(jnp.where(first, revT, 0))
        cc2 = cc + jnp.sum(T)
        return (cc2,
                jnp.where(hit, new_dig, dig),
                jnp.where(hit, rank - above, rin),
                jnp.where(hit, new_bcnt, bcnt))

    z = jnp.int32(0)
    cc, dig, rin, bcnt = lax.fori_loop(
        0, 16, body, (z, jnp.int32(-1), z, z))
    return dig, rin, bcnt


def _sc_select_body(k_active, n, scores_hbm, thr_hbm,
                    data_v, key_v, lista_v, hist_v, totals_v, thr_v):
    nv = n // L
    lane = lax.iota(jnp.int32, L)
    ones = jnp.ones((L,), jnp.int32)
    zeros = jnp.zeros((L,), jnp.int32)
    c = lax.axis_index("c")
    s = lax.axis_index("s")
    wid = s * 2 + c
    rpw = 32

    def zero_hist(_, unused):
        def zb(i, u):
            plsc.store_scatter(hist_v, [lane * 256 + i], zeros)
            return u
        return lax.fori_loop(0, 256, zb, unused)

    def do_row(r, unused):
        row = wid * rpw + r
        pltpu.sync_copy(scores_hbm.at[row], data_v)

        zero_hist(0, 0)

        def l1(i, u):
            base = i * L + lane
            v = plsc.load_gather(data_v, [base])
            b = plsc.bitcast(v, jnp.int32)
            ks = _signed_key(b)
            plsc.store_scatter(key_v, [base], ks)
            d = (lax.shift_right_logical(ks, 24) & 0xFF) ^ 0x80
            plsc.addupdate_scatter(hist_v, [lane * 256 + d], ones)
            return u
        lax.fori_loop(0, nv, l1, 0)

        def red(g, u):
            acc = zeros
            for l in range(L):
                acc = acc + plsc.load_gather(hist_v, [l * 256 + g * L + lane])
            plsc.store_scatter(totals_v, [g * L + lane], acc)
            return u
        lax.fori_loop(0, 16, red, 0)

        d1, rank2, _n1 = _walk(totals_v, jnp.int32(k_active))

        def l2(i, off):
            base = i * L + lane
            ks = plsc.load_gather(key_v, [base])
            b0 = (lax.shift_right_logical(ks, 24) & 0xFF) ^ 0x80
            m = b0 == d1
            cs = plsc.cumsum(m.astype(jnp.int32))
            plsc.store_scatter(lista_v, [off + cs - 1], ks, mask=m)
            return off + jnp.sum(m.astype(jnp.int32))
        n1 = lax.fori_loop(0, nv, l2, jnp.int32(0))

        nv1 = (n1 + (L - 1)) // L

        def bit_step(bi, t):
            bit = jnp.int32(23) - bi
            cand = t | lax.shift_left(jnp.int32(1), bit)

            def cnt_body(i, cnt):
                base = i * L + lane
                kv = plsc.load_gather(lista_v, [base])
                low = kv & jnp.int32(0x00FFFFFF)
                ok = jnp.logical_and(low >= cand, base < n1)
                return cnt + jnp.sum(ok.astype(jnp.int32))
            cnt = lax.fori_loop(0, nv1, cnt_body, jnp.int32(0))
            return jnp.where(cnt >= rank2, cand, t)

        t_low = lax.fori_loop(0, 24, bit_step, jnp.int32(0))
        kth = lax.shift_left((d1 ^ 0x80) & 0xFF, 24) | t_low

        plsc.store_scatter(thr_v, [jnp.full((L,), r, jnp.int32)],
                           jnp.full((L,), kth, jnp.int32), mask=lane == 0)
        return unused

    lax.fori_loop(0, rpw, do_row, 0)
    pltpu.sync_copy(thr_v, thr_hbm.at[pl.ds(wid * rpw, rpw)])


def _select_thresholds(scores_t, k_active):
    O, B = scores_t.shape
    mesh = plsc.VectorSubcoreMesh(core_axis_name="c", subcore_axis_name="s")
    f = pl.kernel(
        functools.partial(_sc_select_body, k_active, B),
        mesh=mesh,
        compiler_params=pltpu.CompilerParams(needs_layout_passes=False),
        out_type=jax.ShapeDtypeStruct((O,), jnp.int32),
        scratch_types=[
            pltpu.VMEM((B,), jnp.float32),
            pltpu.VMEM((B,), jnp.int32),
            pltpu.VMEM((B,), jnp.int32),
            pltpu.VMEM((L * 256,), jnp.int32),
            pltpu.VMEM((256,), jnp.int32),
            pltpu.VMEM((32,), jnp.int32),
        ],
    )
    return f(scores_t)


def _mask_body(x_ref, w_ref, b_ref, t_ref, o_ref):
    x = x_ref[...]
    w = w_ref[...]
    sc = lax.dot_general(
        x, w, (((1,), (1,)), ((), ())), preferred_element_type=jnp.float32)
    ks = _signed_key(lax.bitcast_convert_type(sc, jnp.int32))
    keep = ks >= t_ref[...]
    o_ref[...] = jnp.where(keep, sc + b_ref[...], jnp.float32(0.0))


def _masked_out(x, w, bias2, thr2):
    B, D = x.shape
    O = w.shape[0]
    CB = 128
    return pl.pallas_call(
        _mask_body,
        grid=(O // CB,),
        in_specs=[
            pl.BlockSpec((B, D), lambda j: (0, 0)),
            pl.BlockSpec((CB, D), lambda j: (j, 0)),
            pl.BlockSpec((1, CB), lambda j: (0, j)),
            pl.BlockSpec((1, CB), lambda j: (0, j)),
        ],
        out_specs=pl.BlockSpec((B, CB), lambda j: (0, j)),
        out_shape=jax.ShapeDtypeStruct((B, O), jnp.float32),
    )(x, w, bias2, thr2)


@jax.jit
def kernel(input, weight, bias):
    B, D = input.shape
    O = weight.shape[0]
    k_active = math.ceil(0.5 * B)
    st = _scores_t(input, weight)
    thr = _select_thresholds(st, k_active)
    return _masked_out(input, weight, bias.reshape(1, O), thr.reshape(1, O))


# speedup vs baseline: 1.1819x; 1.1819x over previous
import functools
import math

import jax
import jax.numpy as jnp
from jax import lax
from jax.experimental import pallas as pl
from jax.experimental.pallas import tpu as pltpu
from jax.experimental.pallas import tpu_sc as plsc

L = 16


def _signed_key(b):
    return b ^ (lax.shift_right_arithmetic(b, 31) & jnp.int32(0x7FFFFFFF))


def _matmul_t_body(x_ref, w_ref, o_ref):
    o_ref[...] = lax.dot_general(
        w_ref[...], x_ref[...], (((1,), (1,)), ((), ())),
        preferred_element_type=jnp.float32)


def _scores_t(x, w):
    B, D = x.shape
    O = w.shape[0]
    CB = 128
    return pl.pallas_call(
        _matmul_t_body,
        grid=(O // CB,),
        in_specs=[
            pl.BlockSpec((B, D), lambda j: (0, 0)),
            pl.BlockSpec((CB, D), lambda j: (j, 0)),
        ],
        out_specs=pl.BlockSpec((CB, B), lambda j: (j, 0)),
        out_shape=jax.ShapeDtypeStruct((O, B), jnp.float32),
    )(x, w)


def _walk(totals_v, rank):
    lane = lax.iota(jnp.int32, L)

    def body(g, carry):
        cc, dig, rin = carry
        gg = 15 - g
        T = plsc.load_gather(totals_v, [gg * L + lane])
        revT = lax.rev(T, (0,))
        csum = plsc.cumsum(revT)
        S = csum + cc
        m = S >= rank
        pc = jnp.sum(m.astype(jnp.int32))
        hit = jnp.logical_and(pc > 0, dig < 0)
        new_dig = gg * L + pc - 1
        above = cc + jnp.sum(jnp.where(m, 0, revT))
        cc2 = cc + jnp.sum(T)
        return (cc2,
                jnp.where(hit, new_dig, dig),
                jnp.where(hit, rank - above, rin))

    z = jnp.int32(0)
    _, dig, rin = lax.fori_loop(0, 16, body, (z, jnp.int32(-1), z))
    return dig, rin


def _sc_select_body(k_active, n, scores_hbm, thr_hbm,
                    data_a, data_b, lista_v, hist_v, totals_v, thr_v,
                    sem_a, sem_b):
    nv = n // L
    lane = lax.iota(jnp.int32, L)
    ones = jnp.ones((L,), jnp.int32)
    zeros = jnp.zeros((L,), jnp.int32)
    c = lax.axis_index("c")
    s = lax.axis_index("s")
    wid = s * 2 + c
    rpw = 32
    row0 = wid * rpw

    def select_row(data_v, r):
        def zb(i, u):
            plsc.store_scatter(hist_v, [lane * 256 + i], zeros)
            return u
        lax.fori_loop(0, 256, zb, 0, unroll=8)

        def l1(i, u):
            v = plsc.load_gather(data_v, [i * L + lane])
            ks = _signed_key(plsc.bitcast(v, jnp.int32))
            d = (lax.shift_right_logical(ks, 24) & 0xFF) ^ 0x80
            plsc.addupdate_scatter(hist_v, [lane * 256 + d], ones)
            return u
        lax.fori_loop(0, nv, l1, 0, unroll=8)

        def red(g, u):
            acc = zeros
            for l in range(L):
                acc = acc + plsc.load_gather(hist_v, [l * 256 + g * L + lane])
            plsc.store_scatter(totals_v, [g * L + lane], acc)
            return u
        lax.fori_loop(0, 16, red, 0)

        d1, rank2 = _walk(totals_v, jnp.int32(k_active))

        cnts = plsc.load_gather(hist_v, [lane * 256 + d1])
        base = plsc.cumsum(cnts) - cnts
        n1 = jnp.sum(cnts)

        def l2(i, off):
            v = plsc.load_gather(data_v, [i * L + lane])
            ks = _signed_key(plsc.bitcast(v, jnp.int32))
            b0 = (lax.shift_right_logical(ks, 24) & 0xFF) ^ 0x80
            m = b0 == d1
            plsc.store_scatter(lista_v, [off], ks, mask=m)
            return off + m.astype(jnp.int32)
        lax.fori_loop(0, nv, l2, base, unroll=8)

        nv1 = (n1 + (L - 1)) // L

        def bit_step(bi, t):
            cand = t | lax.shift_left(jnp.int32(1), jnp.int32(23) - bi)

            def cnt_body(i, cntv):
                kv = plsc.load_gather(lista_v, [i * L + lane])
                low = kv & jnp.int32(0x00FFFFFF)
                ok = jnp.logical_and(low >= cand, i * L + lane < n1)
                return cntv + ok.astype(jnp.int32)
            cntv = lax.fori_loop(0, nv1, cnt_body, zeros)
            return jnp.where(jnp.sum(cntv) >= rank2, cand, t)

        t_low = lax.fori_loop(0, 24, bit_step, jnp.int32(0))
        kth = lax.shift_left((d1 ^ 0x80) & 0xFF, 24) | t_low

        plsc.store_scatter(thr_v, [jnp.full((L,), r, jnp.int32)],
                           jnp.full((L,), kth, jnp.int32), mask=lane == 0)

    pltpu.make_async_copy(scores_hbm.at[row0], data_a, sem_a).start()
    pltpu.make_async_copy(scores_hbm.at[row0 + 1], data_b, sem_b).start()

    def do_pair(m, u):
        r = 2 * m
        pltpu.make_async_copy(scores_hbm.at[row0], data_a, sem_a).wait()
        select_row(data_a, r)
        nxt = row0 + lax.rem(r + 2, rpw)
        pltpu.make_async_copy(scores_hbm.at[nxt], data_a, sem_a).start()
        pltpu.make_async_copy(scores_hbm.at[row0 + 1], data_b, sem_b).wait()
        select_row(data_b, r + 1)
        nxt2 = row0 + lax.rem(r + 3, rpw)
        pltpu.make_async_copy(scores_hbm.at[nxt2], data_b, sem_b).start()
        return u
    lax.fori_loop(0, rpw // 2, do_pair, 0)

    pltpu.make_async_copy(scores_hbm.at[row0], data_a, sem_a).wait()
    pltpu.make_async_copy(scores_hbm.at[row0 + 1], data_b, sem_b).wait()

    pltpu.sync_copy(thr_v, thr_hbm.at[pl.ds(row0, rpw)])


def _select_thresholds(scores_t, k_active):
    O, B = scores_t.shape
    mesh = plsc.VectorSubcoreMesh(core_axis_name="c", subcore_axis_name="s")
    f = pl.kernel(
        functools.partial(_sc_select_body, k_active, B),
        mesh=mesh,
        compiler_params=pltpu.CompilerParams(needs_layout_passes=False),
        out_type=jax.ShapeDtypeStruct((O,), jnp.int32),
        scratch_types=[
            pltpu.VMEM((B,), jnp.float32),
            pltpu.VMEM((B,), jnp.float32),
            pltpu.VMEM((B,), jnp.int32),
            pltpu.VMEM((L * 256,), jnp.int32),
            pltpu.VMEM((256,), jnp.int32),
            pltpu.VMEM((32,), jnp.int32),
            pltpu.SemaphoreType.DMA,
            pltpu.SemaphoreType.DMA,
        ],
    )
    return f(scores_t)


def _mask_body(x_ref, w_ref, b_ref, t_ref, o_ref):
    x = x_ref[...]
    w = w_ref[...]
    sc = lax.dot_general(
        x, w, (((1,), (1,)), ((), ())), preferred_element_type=jnp.float32)
    ks = _signed_key(lax.bitcast_convert_type(sc, jnp.int32))
    keep = ks >= t_ref[...]
    o_ref[...] = jnp.where(keep, sc + b_ref[...], jnp.float32(0.0))


def _masked_out(x, w, bias2, thr2):
    B, D = x.shape
    O = w.shape[0]
    CB = 128
    return pl.pallas_call(
        _mask_body,
        grid=(O // CB,),
        in_specs=[
            pl.BlockSpec((B, D), lambda j: (0, 0)),
            pl.BlockSpec((CB, D), lambda j: (j, 0)),
            pl.BlockSpec((1, CB), lambda j: (0, j)),
            pl.BlockSpec((1, CB), lambda j: (0, j)),
        ],
        out_specs=pl.BlockSpec((B, CB), lambda j: (0, j)),
        out_shape=jax.ShapeDtypeStruct((B, O), jnp.float32),
    )(x, w, bias2, thr2)


@jax.jit
def kernel(input, weight, bias):
    B, D = input.shape
    O = weight.shape[0]
    k_active = math.ceil(0.5 * B)
    st = _scores_t(input, weight)
    thr = _select_thresholds(st, k_active)
    return _masked_out(input, weight, bias.reshape(1, O), thr.reshape(1, O))


# speedup vs baseline: 3.1403x; 2.6571x over previous
import functools
import math

import jax
import jax.numpy as jnp
from jax import lax
from jax.experimental import pallas as pl
from jax.experimental.pallas import tpu as pltpu
from jax.experimental.pallas import tpu_sc as plsc

L = 16


def _signed_key(b):
    return b ^ (lax.shift_right_arithmetic(b, 31) & jnp.int32(0x7FFFFFFF))


def _matmul_t_body(x_ref, w_ref, o_ref):
    o_ref[...] = lax.dot_general(
        w_ref[...], x_ref[...], (((1,), (1,)), ((), ())),
        preferred_element_type=jnp.float32)


def _scores_t(x, w):
    B, D = x.shape
    O = w.shape[0]
    CB = 128
    return pl.pallas_call(
        _matmul_t_body,
        grid=(O // CB,),
        in_specs=[
            pl.BlockSpec((B, D), lambda j: (0, 0)),
            pl.BlockSpec((CB, D), lambda j: (j, 0)),
        ],
        out_specs=pl.BlockSpec((CB, B), lambda j: (j, 0)),
        out_shape=jax.ShapeDtypeStruct((O, B), jnp.float32),
    )(x, w)


def _walk(totals_v, rank):
    lane = lax.iota(jnp.int32, L)

    def body(g, carry):
        cc, dig, rin = carry
        gg = 15 - g
        T = plsc.load_gather(totals_v, [gg * L + lane])
        revT = lax.rev(T, (0,))
        csum = plsc.cumsum(revT)
        S = csum + cc
        m = S >= rank
        pc = jnp.sum(m.astype(jnp.int32))
        hit = jnp.logical_and(pc > 0, dig < 0)
        new_dig = gg * L + pc - 1
        above = cc + jnp.sum(jnp.where(m, 0, revT))
        cc2 = cc + jnp.sum(T)
        return (cc2,
                jnp.where(hit, new_dig, dig),
                jnp.where(hit, rank - above, rin))

    z = jnp.int32(0)
    _, dig, rin = lax.fori_loop(0, 16, body, (z, jnp.int32(-1), z))
    return dig, rin


def _sc_select_body(k_active, n, scores_hbm, thr_hbm,
                    data_a, data_b, lista_v, hist_v, totals_v, thr_v,
                    sem_a, sem_b):
    nv = n // L
    lane = lax.iota(jnp.int32, L)
    ones = jnp.ones((L,), jnp.int32)
    zeros = jnp.zeros((L,), jnp.int32)
    c = lax.axis_index("c")
    s = lax.axis_index("s")
    wid = s * 2 + c
    rpw = 32
    row0 = wid * rpw

    lane256 = lane * 256
    mmin = jnp.int32(-2147483648)

    def biased_key(v):
        b = plsc.bitcast(v, jnp.int32)
        sr = lax.shift_right_arithmetic(b, 31)
        return b ^ (sr | mmin)

    def select_row(data_v, r):
        @plsc.parallel_loop(0, nv, unroll=8)
        def l1(i):
            uk = biased_key(plsc.load_gather(data_v, [i * L + lane]))
            d = lax.shift_right_logical(uk, 24)
            plsc.addupdate_scatter(hist_v, [lane256 | d], ones)

        @plsc.parallel_loop(0, 16, unroll=2)
        def red(g):
            acc = zeros
            for l in range(L):
                acc = acc + plsc.load_gather(hist_v, [l * 256 + g * L + lane])
            plsc.store_scatter(totals_v, [g * L + lane], acc)

        d1, rank2 = _walk(totals_v, jnp.int32(k_active))

        cnts = plsc.load_gather(hist_v, [lane256 | d1])
        base = plsc.cumsum(cnts) - cnts
        n1 = jnp.sum(cnts)

        @plsc.parallel_loop(0, 256, unroll=8)
        def zb(i):
            plsc.store_scatter(hist_v, [lane256 | i], zeros)

        @plsc.parallel_loop(0, nv, unroll=8, carry=base)
        def l2(i, off):
            uk = biased_key(plsc.load_gather(data_v, [i * L + lane]))
            m = lax.shift_right_logical(uk, 24) == d1
            plsc.store_scatter(lista_v, [off], uk, mask=m)
            return off + m.astype(jnp.int32)

        nv1 = (n1 + (L - 1)) // L

        def bit_step(bi, t):
            cand = t | lax.shift_left(jnp.int32(1), jnp.int32(23) - bi)

            @plsc.parallel_loop(0, nv1, carry=zeros)
            def cnt_loop(i, cntv):
                kv = plsc.load_gather(lista_v, [i * L + lane])
                low = kv & jnp.int32(0x00FFFFFF)
                ok = jnp.logical_and(low >= cand, i * L + lane < n1)
                return cntv + ok.astype(jnp.int32)
            return jnp.where(jnp.sum(cnt_loop) >= rank2, cand, t)

        t_low = lax.fori_loop(0, 24, bit_step, jnp.int32(0))
        kth = lax.shift_left(d1, 24) | t_low
        kth_s = kth ^ mmin

        plsc.store_scatter(thr_v, [jnp.full((L,), r, jnp.int32)],
                           jnp.full((L,), kth_s, jnp.int32), mask=lane == 0)

    @plsc.parallel_loop(0, 256, unroll=8)
    def zb0(i):
        plsc.store_scatter(hist_v, [lane256 | i], zeros)

    pltpu.make_async_copy(scores_hbm.at[row0], data_a, sem_a).start()
    pltpu.make_async_copy(scores_hbm.at[row0 + 1], data_b, sem_b).start()

    def do_pair(m, u):
        r = 2 * m
        pltpu.make_async_copy(scores_hbm.at[row0], data_a, sem_a).wait()
        select_row(data_a, r)
        nxt = row0 + lax.rem(r + 2, rpw)
        pltpu.make_async_copy(scores_hbm.at[nxt], data_a, sem_a).start()
        pltpu.make_async_copy(scores_hbm.at[row0 + 1], data_b, sem_b).wait()
        select_row(data_b, r + 1)
        nxt2 = row0 + lax.rem(r + 3, rpw)
        pltpu.make_async_copy(scores_hbm.at[nxt2], data_b, sem_b).start()
        return u
    lax.fori_loop(0, rpw // 2, do_pair, 0)

    pltpu.make_async_copy(scores_hbm.at[row0], data_a, sem_a).wait()
    pltpu.make_async_copy(scores_hbm.at[row0 + 1], data_b, sem_b).wait()

    pltpu.sync_copy(thr_v, thr_hbm.at[pl.ds(row0, rpw)])


def _select_thresholds(scores_t, k_active):
    O, B = scores_t.shape
    mesh = plsc.VectorSubcoreMesh(core_axis_name="c", subcore_axis_name="s")
    f = pl.kernel(
        functools.partial(_sc_select_body, k_active, B),
        mesh=mesh,
        compiler_params=pltpu.CompilerParams(needs_layout_passes=False),
        out_type=jax.ShapeDtypeStruct((O,), jnp.int32),
        scratch_types=[
            pltpu.VMEM((B,), jnp.float32),
            pltpu.VMEM((B,), jnp.float32),
            pltpu.VMEM((B,), jnp.int32),
            pltpu.VMEM((L * 256,), jnp.int32),
            pltpu.VMEM((256,), jnp.int32),
            pltpu.VMEM((32,), jnp.int32),
            pltpu.SemaphoreType.DMA,
            pltpu.SemaphoreType.DMA,
        ],
    )
    return f(scores_t)


def _mask_body(x_ref, w_ref, b_ref, t_ref, o_ref):
    x = x_ref[...]
    w = w_ref[...]
    sc = lax.dot_general(
        x, w, (((1,), (1,)), ((), ())), preferred_element_type=jnp.float32)
    ks = _signed_key(lax.bitcast_convert_type(sc, jnp.int32))
    keep = ks >= t_ref[...]
    o_ref[...] = jnp.where(keep, sc + b_ref[...], jnp.float32(0.0))


def _masked_out(x, w, bias2, thr2):
    B, D = x.shape
    O = w.shape[0]
    CB = 128
    return pl.pallas_call(
        _mask_body,
        grid=(O // CB,),
        in_specs=[
            pl.BlockSpec((B, D), lambda j: (0, 0)),
            pl.BlockSpec((CB, D), lambda j: (j, 0)),
            pl.BlockSpec((1, CB), lambda j: (0, j)),
            pl.BlockSpec((1, CB), lambda j: (0, j)),
        ],
        out_specs=pl.BlockSpec((B, CB), lambda j: (0, j)),
        out_shape=jax.ShapeDtypeStruct((B, O), jnp.float32),
    )(x, w, bias2, thr2)


@jax.jit
def kernel(input, weight, bias):
    B, D = input.shape
    O = weight.shape[0]
    k_active = math.ceil(0.5 * B)
    st = _scores_t(input, weight)
    thr = _select_thresholds(st, k_active)
    return _masked_out(input, weight, bias.reshape(1, O), thr.reshape(1, O))
